# baseline (speedup 1.0000x reference)
import functools

import jax
import jax.numpy as jnp
from jax import lax
from jax.experimental import pallas as pl
from jax.experimental.pallas import tpu as pltpu
from jax.experimental.pallas import tpu_sc as plsc

NC = 2
NS = 16
L = 16
NW = NC * NS
B = 16384
D = 128
BPW = B // NW
CH = 128
NCHUNK = BPW // CH
GROUPS = CH // L


def _nsqrt(x):
    i = plsc.bitcast(x, jnp.int32)
    i = jnp.int32(0x1FBD1DF5) + lax.shift_right_logical(i, jnp.int32(1))
    y = plsc.bitcast(i, jnp.float32)
    for _ in range(3):
        y = 0.5 * (y + x / y)
    return y


@functools.partial(
    pl.kernel,
    out_type=jax.ShapeDtypeStruct((B,), jnp.float32),
    mesh=plsc.VectorSubcoreMesh(
        core_axis_name="c", subcore_axis_name="s", num_cores=NC, num_subcores=NS
    ),
    compiler_params=pltpu.CompilerParams(needs_layout_passes=False, skip_device_barrier=True),
    scratch_types=[
        pltpu.VMEM((BPW * 3,), jnp.int32),
        pltpu.VMEM((3, NCHUNK, CH), jnp.int32),
        [pltpu.VMEM((CH, D), jnp.float32) for _ in range(2)],
        [pltpu.VMEM((CH, D), jnp.float32) for _ in range(2)],
        [pltpu.VMEM((CH, D), jnp.float32) for _ in range(2)],
        pltpu.VMEM((CH * L,), jnp.float32),
        pltpu.VMEM((BPW,), jnp.float32),
        [pltpu.SemaphoreType.DMA for _ in range(2)],
    ],
)
def _sc_score(
    idx_hbm, ent_hbm, rel_hbm, out_hbm, raw_v, idx_v, bhs, brs, bts, pvec, outv,
    sems,
):
    c = lax.axis_index("c")
    s = lax.axis_index("s")
    wid = s * NC + c
    pltpu.sync_copy(idx_hbm.at[pl.ds(wid * BPW * 3, BPW * 3)], raw_v)
    lane = lax.iota(jnp.int32, L)
    for o in range(3):
        for k in range(BPW // L):
            src = 3 * (k * L + lane) + o
            vals = plsc.load_gather(raw_v, [src])
            idx_v[o, k // (CH // L), pl.ds((k % (CH // L)) * L, L)] = vals

    def fire(ch):
        b = ch % 2
        return (
            pltpu.async_copy(ent_hbm.at[idx_v.at[0, ch]], bhs[b], sems[b]),
            pltpu.async_copy(rel_hbm.at[idx_v.at[1, ch]], brs[b], sems[b]),
            pltpu.async_copy(ent_hbm.at[idx_v.at[2, ch]], bts[b], sems[b]),
        )

    pending = fire(0)
    for ch in range(NCHUNK):
        for cp in pending:
            cp.wait()
        if ch + 1 < NCHUNK:
            pending = fire(ch + 1)
        bh, br, bt = bhs[ch % 2], brs[ch % 2], bts[ch % 2]

        def item(i, carry, bh=bh, br=br, bt=bt):
            acc = jnp.zeros((L,), jnp.float32)
            for j in range(D // L):
                h = bh[i, pl.ds(j * L, L)]
                r = br[i, pl.ds(j * L, L)]
                t = bt[i, pl.ds(j * L, L)]
                d = (h + r) - t
                acc = acc + d * d
            pvec[pl.ds(i * L, L)] = acc
            return carry

        lax.fori_loop(0, CH, item, 0)

        def group(g, carry, ch=ch):
            base = (g * L + lax.iota(jnp.int32, L)) * L
            tot = jnp.zeros((L,), jnp.float32)
            for j in range(L):
                tot = tot + plsc.load_gather(pvec, [base + j])
            score = -_nsqrt(tot)
            oidx = ch * CH + g * L + lax.iota(jnp.int32, L)
            plsc.store_scatter(outv, [oidx], score)
            return carry

        lax.fori_loop(0, GROUPS, group, 0)
    pltpu.sync_copy(outv, out_hbm.at[pl.ds(wid * BPW, BPW)])


def kernel(batch, ent_embs, rel_embs):
    idx = batch.astype(jnp.int32).reshape(-1)
    scores = _sc_score(idx, ent_embs, rel_embs)
    return scores.reshape(-1, 1)


# speedup vs baseline: 1.1342x; 1.1342x over previous
import functools

import jax
import jax.numpy as jnp
from jax import lax
from jax.experimental import pallas as pl
from jax.experimental.pallas import tpu as pltpu
from jax.experimental.pallas import tpu_sc as plsc

NC = 2
NS = 16
L = 16
NW = NC * NS
B = 16384
D = 128
BPW = B // NW
CH = 64
NCHUNK = BPW // CH
GROUPS = CH // L


def _nsqrt(x):
    i = plsc.bitcast(x, jnp.int32)
    i = jnp.int32(0x1FBD1DF5) + lax.shift_right_logical(i, jnp.int32(1))
    y = plsc.bitcast(i, jnp.float32)
    for _ in range(3):
        y = 0.5 * (y + x / y)
    return y


@functools.partial(
    pl.kernel,
    out_type=jax.ShapeDtypeStruct((B,), jnp.float32),
    mesh=plsc.VectorSubcoreMesh(
        core_axis_name="c", subcore_axis_name="s", num_cores=NC, num_subcores=NS
    ),
    compiler_params=pltpu.CompilerParams(needs_layout_passes=False),
    scratch_types=[
        [pltpu.VMEM((CH, 3), jnp.int32) for _ in range(2)],
        [pltpu.VMEM((3, CH), jnp.int32) for _ in range(2)],
        [pltpu.VMEM((CH, D), jnp.float32) for _ in range(2)],
        [pltpu.VMEM((CH, D), jnp.float32) for _ in range(2)],
        [pltpu.VMEM((CH, D), jnp.float32) for _ in range(2)],
        pltpu.VMEM((CH,), jnp.float32),
        [pltpu.SemaphoreType.DMA for _ in range(2)],
        [pltpu.SemaphoreType.DMA for _ in range(2)],
    ],
)
def _sc_score(
    idx_hbm, ent_hbm, rel_hbm, out_hbm,
    raws, idxs, bhs, brs, bts, outc, sems, rsems,
):
    c = lax.axis_index("c")
    s = lax.axis_index("s")
    wid = s * NC + c
    base = wid * BPW
    lane = lax.iota(jnp.int32, L)

    def fire_raw(ch):
        b = ch % 2
        return pltpu.async_copy(
            idx_hbm.at[pl.ds(base + ch * CH, CH)], raws[b], rsems[b]
        )

    def deinterleave(ch):
        b = ch % 2
        for o in range(3):
            col = jnp.full((L,), o, jnp.int32)
            for k in range(GROUPS):
                vals = plsc.load_gather(raws[b], [k * L + lane, col])
                idxs[b][o, pl.ds(k * L, L)] = vals

    def fire_rows(ch):
        b = ch % 2
        return (
            pltpu.async_copy(ent_hbm.at[idxs[b].at[0]], bhs[b], sems[b]),
            pltpu.async_copy(rel_hbm.at[idxs[b].at[1]], brs[b], sems[b]),
            pltpu.async_copy(ent_hbm.at[idxs[b].at[2]], bts[b], sems[b]),
        )

    raw_pend = {0: fire_raw(0)}
    if NCHUNK > 1:
        raw_pend[1] = fire_raw(1)
    raw_pend[0].wait()
    deinterleave(0)
    pending = fire_rows(0)

    for ch in range(NCHUNK):
        for cp in pending:
            cp.wait()
        if ch + 2 < NCHUNK:
            raw_pend[ch + 2] = fire_raw(ch + 2)
        if ch + 1 < NCHUNK:
            raw_pend[ch + 1].wait()
            deinterleave(ch + 1)
            pending = fire_rows(ch + 1)
        b = ch % 2
        bh, br, bt = bhs[b], brs[b], bts[b]

        def item(i, carry, bh=bh, br=br, bt=bt):
            acc = jnp.zeros((L,), jnp.float32)
            for j in range(D // L):
                h = bh[i, pl.ds(j * L, L)]
                r = br[i, pl.ds(j * L, L)]
                t = bt[i, pl.ds(j * L, L)]
                d = (h + r) - t
                acc = acc + d * d
            bh[i, pl.ds(0, L)] = acc
            return carry

        lax.fori_loop(0, CH, item, 0)

        def group(g, carry, bh=bh):
            rows = g * L + lane
            tot = jnp.zeros((L,), jnp.float32)
            for j in range(L):
                tot = tot + plsc.load_gather(bh, [rows, jnp.full((L,), j, jnp.int32)])
            score = -_nsqrt(tot)
            plsc.store_scatter(outc, [g * L + lane], score)
            return carry

        lax.fori_loop(0, GROUPS, group, 0)
        pltpu.sync_copy(outc, out_hbm.at[pl.ds(base + ch * CH, CH)])


def kernel(batch, ent_embs, rel_embs):
    scores = _sc_score(batch.astype(jnp.int32), ent_embs, rel_embs)
    return scores.reshape(-1, 1)


# speedup vs baseline: 1.2298x; 1.0843x over previous
import functools

import jax
import jax.numpy as jnp
from jax import lax
from jax.experimental import pallas as pl
from jax.experimental.pallas import tpu as pltpu
from jax.experimental.pallas import tpu_sc as plsc

NC = 2
NS = 16
L = 16
NW = NC * NS
B = 16384
D = 128
BPW = B // NW
CH = 64
NCHUNK = BPW // CH
GROUPS = CH // L


def _nsqrt(x):
    i = plsc.bitcast(x, jnp.int32)
    i = jnp.int32(0x1FBD1DF5) + lax.shift_right_logical(i, jnp.int32(1))
    y = plsc.bitcast(i, jnp.float32)
    for _ in range(3):
        y = 0.5 * (y + x / y)
    return y


@functools.partial(
    pl.kernel,
    out_type=jax.ShapeDtypeStruct((B,), jnp.float32),
    mesh=plsc.VectorSubcoreMesh(
        core_axis_name="c", subcore_axis_name="s", num_cores=NC, num_subcores=NS
    ),
    compiler_params=pltpu.CompilerParams(needs_layout_passes=False),
    scratch_types=[
        pltpu.VMEM((BPW,), jnp.int32),
        pltpu.VMEM((BPW,), jnp.int32),
        pltpu.VMEM((BPW,), jnp.int32),
        [pltpu.VMEM((CH, D), jnp.float32) for _ in range(2)],
        [pltpu.VMEM((CH, D), jnp.float32) for _ in range(2)],
        [pltpu.VMEM((CH, D), jnp.float32) for _ in range(2)],
        pltpu.VMEM((CH,), jnp.float32),
        [pltpu.SemaphoreType.DMA for _ in range(2)],
        pltpu.SemaphoreType.DMA,
    ],
)
def _sc_score(
    hs_hbm, rs_hbm, ts_hbm, ent_hbm, rel_hbm, out_hbm,
    hv, rv, tv, bhs, brs, bts, outc, sems, isem,
):
    c = lax.axis_index("c")
    s = lax.axis_index("s")
    wid = s * NC + c
    base = wid * BPW
    lane = lax.iota(jnp.int32, L)

    ih = pltpu.async_copy(hs_hbm.at[pl.ds(base, BPW)], hv, isem)
    ir = pltpu.async_copy(rs_hbm.at[pl.ds(base, BPW)], rv, isem)
    it = pltpu.async_copy(ts_hbm.at[pl.ds(base, BPW)], tv, isem)
    ih.wait()
    ir.wait()
    it.wait()

    def fire_rows(ch):
        b = ch % 2
        sl = pl.ds(ch * CH, CH)
        return (
            pltpu.async_copy(ent_hbm.at[hv.at[sl]], bhs[b], sems[b]),
            pltpu.async_copy(rel_hbm.at[rv.at[sl]], brs[b], sems[b]),
            pltpu.async_copy(ent_hbm.at[tv.at[sl]], bts[b], sems[b]),
        )

    pending = fire_rows(0)
    for ch in range(NCHUNK):
        for cp in pending:
            cp.wait()
        if ch + 1 < NCHUNK:
            pending = fire_rows(ch + 1)
        b = ch % 2
        bh, br, bt = bhs[b], brs[b], bts[b]

        def item(i, carry, bh=bh, br=br, bt=bt):
            acc = jnp.zeros((L,), jnp.float32)
            for j in range(D // L):
                h = bh[i, pl.ds(j * L, L)]
                r = br[i, pl.ds(j * L, L)]
                t = bt[i, pl.ds(j * L, L)]
                d = (h + r) - t
                acc = acc + d * d
            bh[i, pl.ds(0, L)] = acc
            return carry

        lax.fori_loop(0, CH, item, 0)

        def group(g, carry, bh=bh):
            rows = g * L + lane
            tot = jnp.zeros((L,), jnp.float32)
            for j in range(L):
                col = jnp.full((L,), j, jnp.int32)
                tot = tot + plsc.load_gather(bh, [rows, col])
            score = -_nsqrt(tot)
            plsc.store_scatter(outc, [g * L + lane], score)
            return carry

        lax.fori_loop(0, GROUPS, group, 0)
        pltpu.sync_copy(outc, out_hbm.at[pl.ds(base + ch * CH, CH)])


def kernel(batch, ent_embs, rel_embs):
    b32 = batch.astype(jnp.int32)
    scores = _sc_score(b32[:, 0], b32[:, 1], b32[:, 2], ent_embs, rel_embs)
    return scores.reshape(-1, 1)
